# baseline (speedup 1.0000x reference)
import functools
import math

import jax
import jax.numpy as jnp
from jax import lax
from jax.experimental import pallas as pl
from jax.experimental.pallas import tpu as pltpu
from jax.experimental.pallas import tpu_sc as plsc

CLASS_DIM = 32
API_DIM = 64
CAT_DIM = CLASS_DIM + API_DIM
S32 = math.sqrt(float(CLASS_DIM))
S96 = math.sqrt(float(CAT_DIM))

NC = 2
NS = 16
NW = NC * NS
LANES = 16


@functools.lru_cache(maxsize=None)
def _make_sc_kernel(N: int, chunk: int):
    per_w = N // NW
    n_chunks = per_w // chunk
    assert per_w % chunk == 0 and chunk % 8 == 0

    mesh = plsc.VectorSubcoreMesh(core_axis_name="c", subcore_axis_name="s")

    @functools.partial(
        pl.kernel,
        mesh=mesh,
        compiler_params=pltpu.CompilerParams(use_tc_tiling_on_sc=False),
        out_type=(
            jax.ShapeDtypeStruct((N, CLASS_DIM), jnp.float32),
            jax.ShapeDtypeStruct((N, CAT_DIM), jnp.float32),
        ),
        scratch_types=[
            pltpu.VMEM((chunk,), jnp.int32),
            pltpu.VMEM((chunk,), jnp.int32),
            pltpu.VMEM((chunk, CLASS_DIM), jnp.float32),
            pltpu.VMEM((chunk, API_DIM), jnp.float32),
            pltpu.VMEM((chunk, CAT_DIM), jnp.float32),
            pltpu.SemaphoreType.DMA,
            pltpu.SemaphoreType.DMA,
        ],
    )
    def k(cls_seq, api_seq, cls_tab, api_tab, out1, out2,
          cidx_v, aidx_v, cls_v, api_v, cat_v, sem1, sem2):
        wid = lax.axis_index("s") * NC + lax.axis_index("c")
        base = wid * per_w

        def chunk_body(j, carry):
            off = base + j * chunk
            pltpu.sync_copy(cls_seq.at[pl.ds(off, chunk)], cidx_v)
            pltpu.sync_copy(api_seq.at[pl.ds(off, chunk)], aidx_v)
            g1 = pltpu.async_copy(cls_tab.at[cidx_v], cls_v, sem1)
            g2 = pltpu.async_copy(api_tab.at[aidx_v], api_v, sem2)
            g1.wait()
            g2.wait()

            def row_body(r, rcarry):
                for h in range(CLASS_DIM // LANES):
                    v = cls_v[r, pl.ds(h * LANES, LANES)]
                    cat_v[r, pl.ds(h * LANES, LANES)] = v * S96
                    cls_v[r, pl.ds(h * LANES, LANES)] = v * S32
                for h in range(API_DIM // LANES):
                    v = api_v[r, pl.ds(h * LANES, LANES)]
                    cat_v[r, pl.ds(CLASS_DIM + h * LANES, LANES)] = v * S96
                return rcarry

            lax.fori_loop(0, chunk, row_body, 0)

            pltpu.sync_copy(cls_v, out1.at[pl.ds(off, chunk)])
            pltpu.sync_copy(cat_v, out2.at[pl.ds(off, chunk)])
            return carry

        lax.fori_loop(0, n_chunks, chunk_body, 0)

    return k


def kernel(class_seq, api_seq, class_table, api_table):
    B, L = class_seq.shape
    N = B * L
    cls_flat = class_seq.reshape(N).astype(jnp.int32)
    api_flat = api_seq.reshape(N).astype(jnp.int32)
    out1, out2 = _make_sc_kernel(N, 640)(cls_flat, api_flat,
                                         class_table, api_table)
    return (out1.reshape(B, L, CLASS_DIM), out2.reshape(B, L, CAT_DIM))
